# Initial kernel scaffold; baseline (speedup 1.0000x reference)
import jax
import jax.numpy as jnp
from jax.experimental import pallas as pl


def kernel(node, edge_index, eps_k):
    raise NotImplementedError("write your pallas kernel here")


# speedup vs baseline: 4.2975x; 4.2975x over previous
import jax
import jax.numpy as jnp
from jax import lax
from jax.experimental import pallas as pl
from jax.experimental.pallas import tpu as pltpu
from jax.experimental.pallas import tpu_sc as plsc

N_NODES = 10000
N_EDGES = 320000
D_FEAT = 128
DH = 64
CHUNK = 128
NS = 16
NC = 2
NPAD = 10240
ROWS_PER_TILE = NPAD // NS
SUB = 320
NSUB = ROWS_PER_TILE // SUB
NCHUNKS = N_EDGES // CHUNK
ITERS = (NCHUNKS + NS - 1) // NS


def _gin_body(table, idx2, dst, eps, out, acc, bufa, bufb, rows, sidx, didx,
              epsv, sem):
    c = lax.axis_index("c")
    s = lax.axis_index("s")
    base_row = c * NPAD + s * ROWS_PER_TILE

    zero16 = jnp.zeros((16,), jnp.float32)

    def zrow(r, carry):
        for c4 in range(DH // 16):
            bufa[r, pl.ds(c4 * 16, 16)] = zero16
        return carry

    lax.fori_loop(0, SUB, zrow, 0)
    for b in range(NSUB):
        pltpu.sync_copy(bufa, acc.at[pl.ds(s * ROWS_PER_TILE + b * SUB, SUB)])
    pltpu.sync_copy(eps, epsv)
    plsc.subcore_barrier()

    def chunk(i, carry):
        j = s + i * NS

        @pl.when(j < NCHUNKS)
        def _():
            off = j * CHUNK
            pltpu.sync_copy(idx2.at[c, pl.ds(off, CHUNK)], sidx)
            pltpu.sync_copy(dst.at[pl.ds(off, CHUNK)], didx)
            pltpu.async_copy(table.at[sidx], rows, sem).wait()
            pltpu.sync_copy(rows, acc.at[didx], add=True)

        return carry

    lax.fori_loop(0, ITERS, chunk, 0)
    plsc.subcore_barrier()

    scale = epsv[...] + 1.0

    def frow(r, carry):
        for c4 in range(DH // 16):
            dsl = pl.ds(c4 * 16, 16)
            bufa[r, dsl] = bufb[r, dsl] * scale + bufa[r, dsl]
        return carry

    for b in range(NSUB):
        pltpu.sync_copy(acc.at[pl.ds(s * ROWS_PER_TILE + b * SUB, SUB)], bufa)
        pltpu.sync_copy(table.at[pl.ds(base_row + b * SUB, SUB)], bufb)
        lax.fori_loop(0, SUB, frow, 0)
        pltpu.sync_copy(bufa, out.at[pl.ds(base_row + b * SUB, SUB)])


def kernel(node, edge_index, eps_k):
    pad = jnp.zeros((NPAD - N_NODES, DH), jnp.float32)
    table = jnp.concatenate([node[:, :DH], pad, node[:, DH:], pad], axis=0)
    src = edge_index[1]
    idx2 = jnp.stack([src, src + NPAD])
    dst = edge_index[0]
    eps = jnp.broadcast_to(jnp.reshape(eps_k.astype(jnp.float32), (1,)), (16,))

    mesh = plsc.VectorSubcoreMesh(core_axis_name="c", subcore_axis_name="s")
    run = pl.kernel(
        _gin_body,
        out_type=jax.ShapeDtypeStruct((NC * NPAD, DH), jnp.float32),
        mesh=mesh,
        compiler_params=pltpu.CompilerParams(use_tc_tiling_on_sc=False),
        scratch_types=[
            pltpu.VMEM_SHARED((NPAD, DH), jnp.float32),
            pltpu.VMEM((SUB, DH), jnp.float32),
            pltpu.VMEM((SUB, DH), jnp.float32),
            pltpu.VMEM((CHUNK, DH), jnp.float32),
            pltpu.VMEM((CHUNK,), jnp.int32),
            pltpu.VMEM((CHUNK,), jnp.int32),
            pltpu.VMEM((16,), jnp.float32),
            pltpu.SemaphoreType.DMA,
        ],
    )
    out_cat = run(table, idx2, dst, eps)
    return jnp.concatenate([out_cat[:N_NODES], out_cat[NPAD:NPAD + N_NODES]], axis=1)


# speedup vs baseline: 4.7364x; 1.1021x over previous
import jax
import jax.numpy as jnp
from jax import lax
from jax.experimental import pallas as pl
from jax.experimental.pallas import tpu as pltpu
from jax.experimental.pallas import tpu_sc as plsc

N_NODES = 10000
N_EDGES = 320000
D_FEAT = 128
DH = 64
CHUNK = 128
NS = 16
NC = 2
NPAD = 10240
ROWS_PER_TILE = NPAD // NS
SUB = 320
NSUB = ROWS_PER_TILE // SUB
GSZ = 8
GROUPS = 20
CPT = GSZ * GROUPS
NCHUNKS = CPT * NS
E_PAD = NCHUNKS * CHUNK


def _gin_body(table, idx3, dst3, eps, out, acc, bufa, bufb, rows, sidx8,
              didx8, epsv, sem0, sem1):
    c = lax.axis_index("c")
    s = lax.axis_index("s")
    base_row = c * NPAD + s * ROWS_PER_TILE
    sems = (sem0, sem1)

    zero16 = jnp.zeros((16,), jnp.float32)

    def zrow(r, carry):
        for c4 in range(DH // 16):
            bufa[r, pl.ds(c4 * 16, 16)] = zero16
        return carry

    lax.fori_loop(0, SUB, zrow, 0)
    for b in range(NSUB):
        pltpu.sync_copy(bufa, acc.at[pl.ds(s * ROWS_PER_TILE + b * SUB, SUB)])
    pltpu.sync_copy(eps, epsv)
    plsc.subcore_barrier()

    def group(g, carry):
        j0 = s * CPT + g * GSZ
        pltpu.sync_copy(idx3.at[c, pl.ds(j0, GSZ)], sidx8)
        pltpu.sync_copy(dst3.at[pl.ds(j0, GSZ)], didx8)
        cps = [pltpu.async_copy(table.at[sidx8.at[0]], rows.at[0], sem0)]
        for q in range(GSZ):
            nq = q + 1
            if nq < GSZ:
                cps.append(pltpu.async_copy(
                    table.at[sidx8.at[nq]], rows.at[nq & 1], sems[nq & 1]))
            cps[q].wait()
            pltpu.sync_copy(rows.at[q & 1], acc.at[didx8.at[q]], add=True)
        return carry

    lax.fori_loop(0, GROUPS, group, 0)
    plsc.subcore_barrier()

    scale = epsv[...] + 1.0

    def frow(r, carry):
        for c4 in range(DH // 16):
            dsl = pl.ds(c4 * 16, 16)
            bufa[r, dsl] = bufb[r, dsl] * scale + bufa[r, dsl]
        return carry

    for b in range(NSUB):
        pltpu.sync_copy(acc.at[pl.ds(s * ROWS_PER_TILE + b * SUB, SUB)], bufa)
        pltpu.sync_copy(table.at[pl.ds(base_row + b * SUB, SUB)], bufb)
        lax.fori_loop(0, SUB, frow, 0)
        pltpu.sync_copy(bufa, out.at[pl.ds(base_row + b * SUB, SUB)])


def kernel(node, edge_index, eps_k):
    pad = jnp.zeros((NPAD - N_NODES, DH), jnp.float32)
    table = jnp.concatenate([node[:, :DH], pad, node[:, DH:], pad], axis=0)
    epad = E_PAD - N_EDGES
    src = jnp.concatenate(
        [edge_index[1], jnp.full((epad,), N_NODES, jnp.int32)])
    dst = jnp.concatenate(
        [edge_index[0], jnp.full((epad,), NPAD - 1, jnp.int32)])
    idx3 = jnp.stack([src, src + NPAD]).reshape(NC, NCHUNKS, CHUNK)
    dst3 = dst.reshape(NCHUNKS, CHUNK)
    eps = jnp.broadcast_to(jnp.reshape(eps_k.astype(jnp.float32), (1,)), (16,))

    mesh = plsc.VectorSubcoreMesh(core_axis_name="c", subcore_axis_name="s")
    run = pl.kernel(
        _gin_body,
        out_type=jax.ShapeDtypeStruct((NC * NPAD, DH), jnp.float32),
        mesh=mesh,
        compiler_params=pltpu.CompilerParams(use_tc_tiling_on_sc=False),
        scratch_types=[
            pltpu.VMEM_SHARED((NPAD, DH), jnp.float32),
            pltpu.VMEM((SUB, DH), jnp.float32),
            pltpu.VMEM((SUB, DH), jnp.float32),
            pltpu.VMEM((2, CHUNK, DH), jnp.float32),
            pltpu.VMEM((GSZ, CHUNK), jnp.int32),
            pltpu.VMEM((GSZ, CHUNK), jnp.int32),
            pltpu.VMEM((16,), jnp.float32),
            pltpu.SemaphoreType.DMA,
            pltpu.SemaphoreType.DMA,
        ],
    )
    out_cat = run(table, idx3, dst3, eps)
    return jnp.concatenate([out_cat[:N_NODES], out_cat[NPAD:NPAD + N_NODES]],
                           axis=1)


# speedup vs baseline: 4.7664x; 1.0063x over previous
import jax
import jax.numpy as jnp
from jax import lax
from jax.experimental import pallas as pl
from jax.experimental.pallas import tpu as pltpu
from jax.experimental.pallas import tpu_sc as plsc

N_NODES = 10000
N_EDGES = 320000
D_FEAT = 128
DH = 64
CHUNK = 128
NS = 16
NC = 2
NPAD = 10240
ROWS_PER_TILE = NPAD // NS
SUB = 320
NSUB = ROWS_PER_TILE // SUB
GSZ = 8
NBUF = 4
GROUPS = 20
CPT = GSZ * GROUPS
NCHUNKS = CPT * NS
E_PAD = NCHUNKS * CHUNK


def _gin_body(table, idx3, dst3, eps, out, acc, bufa, bufb, rows, sidx8,
              didx8, epsv, gs0, gs1, gs2, gs3, ss0, ss1, ss2, ss3):
    c = lax.axis_index("c")
    s = lax.axis_index("s")
    base_row = c * NPAD + s * ROWS_PER_TILE
    gsems = (gs0, gs1, gs2, gs3)
    ssems = (ss0, ss1, ss2, ss3)

    zero16 = jnp.zeros((16,), jnp.float32)

    def zrow(r, carry):
        for c4 in range(DH // 16):
            bufa[r, pl.ds(c4 * 16, 16)] = zero16
        return carry

    lax.fori_loop(0, SUB, zrow, 0)
    for b in range(NSUB):
        pltpu.sync_copy(bufa, acc.at[pl.ds(s * ROWS_PER_TILE + b * SUB, SUB)])
    pltpu.sync_copy(eps, epsv)
    plsc.subcore_barrier()

    def group(g, carry):
        j0 = s * CPT + g * GSZ
        pltpu.sync_copy(idx3.at[c, pl.ds(j0, GSZ)], sidx8)
        pltpu.sync_copy(dst3.at[pl.ds(j0, GSZ)], didx8)
        gcp = {}
        scp = {}
        for q in range(GSZ):
            slot = q % NBUF
            if q >= NBUF:
                scp[q - NBUF].wait()
            gcp[q] = pltpu.async_copy(
                table.at[sidx8.at[q]], rows.at[slot], gsems[slot])
            if q >= 1:
                p = q - 1
                gcp[p].wait()
                scp[p] = pltpu.async_copy(
                    rows.at[p % NBUF], acc.at[didx8.at[p]], ssems[p % NBUF],
                    add=True)
        gcp[GSZ - 1].wait()
        scp[GSZ - 1] = pltpu.async_copy(
            rows.at[(GSZ - 1) % NBUF], acc.at[didx8.at[GSZ - 1]],
            ssems[(GSZ - 1) % NBUF], add=True)
        for q in range(GSZ - NBUF, GSZ):
            scp[q].wait()
        return carry

    lax.fori_loop(0, GROUPS, group, 0)
    plsc.subcore_barrier()

    scale = epsv[...] + 1.0

    def frow(r, carry):
        for c4 in range(DH // 16):
            dsl = pl.ds(c4 * 16, 16)
            bufa[r, dsl] = bufb[r, dsl] * scale + bufa[r, dsl]
        return carry

    for b in range(NSUB):
        pltpu.sync_copy(acc.at[pl.ds(s * ROWS_PER_TILE + b * SUB, SUB)], bufa)
        pltpu.sync_copy(table.at[pl.ds(base_row + b * SUB, SUB)], bufb)
        lax.fori_loop(0, SUB, frow, 0)
        pltpu.sync_copy(bufa, out.at[pl.ds(base_row + b * SUB, SUB)])


def kernel(node, edge_index, eps_k):
    pad = jnp.zeros((NPAD - N_NODES, DH), jnp.float32)
    table = jnp.concatenate([node[:, :DH], pad, node[:, DH:], pad], axis=0)
    epad = E_PAD - N_EDGES
    src = jnp.concatenate(
        [edge_index[1], jnp.full((epad,), N_NODES, jnp.int32)])
    dst = jnp.concatenate(
        [edge_index[0], jnp.full((epad,), NPAD - 1, jnp.int32)])
    idx3 = jnp.stack([src, src + NPAD]).reshape(NC, NCHUNKS, CHUNK)
    dst3 = dst.reshape(NCHUNKS, CHUNK)
    eps = jnp.broadcast_to(jnp.reshape(eps_k.astype(jnp.float32), (1,)), (16,))

    mesh = plsc.VectorSubcoreMesh(core_axis_name="c", subcore_axis_name="s")
    run = pl.kernel(
        _gin_body,
        out_type=jax.ShapeDtypeStruct((NC * NPAD, DH), jnp.float32),
        mesh=mesh,
        compiler_params=pltpu.CompilerParams(use_tc_tiling_on_sc=False),
        scratch_types=[
            pltpu.VMEM_SHARED((NPAD, DH), jnp.float32),
            pltpu.VMEM((SUB, DH), jnp.float32),
            pltpu.VMEM((SUB, DH), jnp.float32),
            pltpu.VMEM((NBUF, CHUNK, DH), jnp.float32),
            pltpu.VMEM((GSZ, CHUNK), jnp.int32),
            pltpu.VMEM((GSZ, CHUNK), jnp.int32),
            pltpu.VMEM((16,), jnp.float32),
        ] + [pltpu.SemaphoreType.DMA] * 8,
    )
    out_cat = run(table, idx3, dst3, eps)
    return jnp.concatenate([out_cat[:N_NODES], out_cat[NPAD:NPAD + N_NODES]],
                           axis=1)


# speedup vs baseline: 4.9846x; 1.0458x over previous
import jax
import jax.numpy as jnp
from jax import lax
from jax.experimental import pallas as pl
from jax.experimental.pallas import tpu as pltpu
from jax.experimental.pallas import tpu_sc as plsc

N_NODES = 10000
N_EDGES = 320000
D_FEAT = 128
DH = 64
CHUNK = 128
NS = 16
NC = 2
NPAD = 10240
ROWS_PER_TILE = NPAD // NS
SUB = 320
NSUB = ROWS_PER_TILE // SUB
GSZ = 8
NBUF = 4
GROUPS = 20
CPT = GSZ * GROUPS
NCHUNKS = CPT * NS
E_PAD = NCHUNKS * CHUNK
FIN_A = 624
FIN_B = N_NODES - FIN_A * (NS - 1)


def _gin_body(table, srcp, dst3, eps, out, acc, bufa, bufb, rows,
              sidx8, didx8, epsv, gs0, gs1, gs2, gs3, ss0, ss1, ss2, ss3):
    c = lax.axis_index("c")
    s = lax.axis_index("s")
    gsems = (gs0, gs1, gs2, gs3)
    ssems = (ss0, ss1, ss2, ss3)
    cvec = jnp.zeros((16,), jnp.int32) + c * NPAD

    zero16 = jnp.zeros((16,), jnp.float32)

    def zrow(r, carry):
        for c4 in range(DH // 16):
            bufa[r, pl.ds(c4 * 16, 16)] = zero16
        return carry

    lax.fori_loop(0, SUB, zrow, 0)
    for b in range(NSUB):
        pltpu.sync_copy(bufa, acc.at[pl.ds(s * ROWS_PER_TILE + b * SUB, SUB)])
    pltpu.sync_copy(eps, epsv)
    plsc.subcore_barrier()

    def group(g, carry):
        j0 = s * CPT + g * GSZ
        pltpu.sync_copy(srcp.at[pl.ds(j0, GSZ)], sidx8)
        pltpu.sync_copy(dst3.at[pl.ds(j0, GSZ)], didx8)
        gcp = {}
        scp = {}
        for q in range(GSZ):
            slot = q % NBUF
            for k in range(CHUNK // 16):
                dsl = pl.ds(k * 16, 16)
                sidx8[q, dsl] = sidx8[q, dsl] + cvec
            if q >= NBUF:
                scp[q - NBUF].wait()
            gcp[q] = pltpu.async_copy(
                table.at[sidx8.at[q]], rows.at[slot], gsems[slot])
            if q >= 1:
                p = q - 1
                gcp[p].wait()
                scp[p] = pltpu.async_copy(
                    rows.at[p % NBUF], acc.at[didx8.at[p]], ssems[p % NBUF],
                    add=True)
        gcp[GSZ - 1].wait()
        scp[GSZ - 1] = pltpu.async_copy(
            rows.at[(GSZ - 1) % NBUF], acc.at[didx8.at[GSZ - 1]],
            ssems[(GSZ - 1) % NBUF], add=True)
        for q in range(GSZ - NBUF, GSZ):
            scp[q].wait()
        return carry

    lax.fori_loop(0, GROUPS, group, 0)
    plsc.subcore_barrier()

    scale = epsv[...] + 1.0
    col0 = pl.multiple_of(c * DH, DH)

    def frow(r, carry):
        for c4 in range(DH // 16):
            dsl = pl.ds(c4 * 16, 16)
            bufa[r, dsl] = bufb[r, dsl] * scale + bufa[r, dsl]
        return carry

    def finalize(row0, nrows):
        nb = nrows // 2
        for b in range(2):
            off = row0 + b * nb
            pltpu.sync_copy(acc.at[pl.ds(off, nb)], bufa.at[pl.ds(0, nb)])
            pltpu.sync_copy(table.at[pl.ds(c * NPAD + off, nb)],
                            bufb.at[pl.ds(0, nb)])
            lax.fori_loop(0, nb, frow, 0)
            pltpu.sync_copy(bufa.at[pl.ds(0, nb)],
                            out.at[pl.ds(off, nb), pl.ds(col0, DH)])

    @pl.when(s < NS - 1)
    def _():
        finalize(s * FIN_A, FIN_A)

    @pl.when(s == NS - 1)
    def _():
        finalize((NS - 1) * FIN_A, FIN_B)


def kernel(node, edge_index, eps_k):
    pad = jnp.zeros((NPAD - N_NODES, DH), jnp.float32)
    table = jnp.concatenate([node[:, :DH], pad, node[:, DH:], pad], axis=0)
    epad = E_PAD - N_EDGES
    srcp = jnp.concatenate(
        [edge_index[1],
         jnp.full((epad,), N_NODES, jnp.int32)]).reshape(NCHUNKS, CHUNK)
    dst3 = jnp.concatenate(
        [edge_index[0],
         jnp.full((epad,), NPAD - 1, jnp.int32)]).reshape(NCHUNKS, CHUNK)
    eps = jnp.broadcast_to(jnp.reshape(eps_k.astype(jnp.float32), (1,)), (16,))

    mesh = plsc.VectorSubcoreMesh(core_axis_name="c", subcore_axis_name="s")
    run = pl.kernel(
        _gin_body,
        out_type=jax.ShapeDtypeStruct((N_NODES, D_FEAT), jnp.float32),
        mesh=mesh,
        compiler_params=pltpu.CompilerParams(use_tc_tiling_on_sc=False),
        scratch_types=[
            pltpu.VMEM_SHARED((NPAD, DH), jnp.float32),
            pltpu.VMEM((SUB, DH), jnp.float32),
            pltpu.VMEM((SUB, DH), jnp.float32),
            pltpu.VMEM((NBUF, CHUNK, DH), jnp.float32),
            pltpu.VMEM((GSZ, CHUNK), jnp.int32),
            pltpu.VMEM((GSZ, CHUNK), jnp.int32),
            pltpu.VMEM((16,), jnp.float32),
        ] + [pltpu.SemaphoreType.DMA] * 8,
    )
    return run(table, srcp, dst3, eps)


# speedup vs baseline: 8.4601x; 1.6972x over previous
import jax
import jax.numpy as jnp
from jax import lax
from jax.experimental import pallas as pl
from jax.experimental.pallas import tpu as pltpu
from jax.experimental.pallas import tpu_sc as plsc

N_NODES = 10000
N_EDGES = 320000
D_FEAT = 128
DH = 64
CHUNK = 128
NS = 16
NC = 2
NPAD = 10240
BLK = 80
NBLK = NPAD // BLK
NOUT = N_NODES // BLK
GSZ = 8
NBUF = 4
GROUPS = 20
CPT = GSZ * GROUPS
NCHUNKS = CPT * NS
E_PAD = NCHUNKS * CHUNK


def _gin_body(table_hbm, srcp, dst3, eps, out, tbl, acc, bufb, rows,
              sidx8, didx8, epsv, gs0, gs1, gs2, gs3, ss0, ss1, ss2, ss3):
    c = lax.axis_index("c")
    s = lax.axis_index("s")
    gsems = (gs0, gs1, gs2, gs3)
    ssems = (ss0, ss1, ss2, ss3)

    pltpu.sync_copy(eps, epsv)
    scale = epsv[...] + 1.0

    def srow(r, carry):
        for c4 in range(DH // 16):
            dsl = pl.ds(c4 * 16, 16)
            bufb[r, dsl] = bufb[r, dsl] * scale
        return carry

    def stage(k, carry):
        off = (s + NS * k) * BLK
        pltpu.sync_copy(table_hbm.at[pl.ds(c * NPAD + off, BLK)], bufb)
        pltpu.sync_copy(bufb, tbl.at[pl.ds(off, BLK)])
        lax.fori_loop(0, BLK, srow, 0)
        pltpu.sync_copy(bufb, acc.at[pl.ds(off, BLK)])
        return carry

    lax.fori_loop(0, NBLK // NS, stage, 0)
    plsc.subcore_barrier()

    def group(g, carry):
        j0 = s * CPT + g * GSZ
        pltpu.sync_copy(srcp.at[pl.ds(j0, GSZ)], sidx8)
        pltpu.sync_copy(dst3.at[pl.ds(j0, GSZ)], didx8)
        gcp = {}
        scp = {}
        for q in range(GSZ):
            slot = q % NBUF
            if q >= NBUF:
                scp[q - NBUF].wait()
            gcp[q] = pltpu.async_copy(
                tbl.at[sidx8.at[q]], rows.at[slot], gsems[slot])
            if q >= 1:
                p = q - 1
                gcp[p].wait()
                scp[p] = pltpu.async_copy(
                    rows.at[p % NBUF], acc.at[didx8.at[p]], ssems[p % NBUF],
                    add=True)
        gcp[GSZ - 1].wait()
        scp[GSZ - 1] = pltpu.async_copy(
            rows.at[(GSZ - 1) % NBUF], acc.at[didx8.at[GSZ - 1]],
            ssems[(GSZ - 1) % NBUF], add=True)
        for q in range(GSZ - NBUF, GSZ):
            scp[q].wait()
        return carry

    lax.fori_loop(0, GROUPS, group, 0)
    plsc.subcore_barrier()

    col0 = pl.multiple_of(c * DH, DH)

    def fout(k, carry):
        blk = s + NS * k

        @pl.when(blk < NOUT)
        def _():
            off = blk * BLK
            pltpu.sync_copy(acc.at[pl.ds(off, BLK)],
                            out.at[pl.ds(off, BLK), pl.ds(col0, DH)])

        return carry

    lax.fori_loop(0, NBLK // NS, fout, 0)


def kernel(node, edge_index, eps_k):
    pad = jnp.zeros((NPAD - N_NODES, DH), jnp.float32)
    table = jnp.concatenate([node[:, :DH], pad, node[:, DH:], pad], axis=0)
    epad = E_PAD - N_EDGES
    srcp = jnp.concatenate(
        [edge_index[1],
         jnp.full((epad,), N_NODES, jnp.int32)]).reshape(NCHUNKS, CHUNK)
    dst3 = jnp.concatenate(
        [edge_index[0],
         jnp.full((epad,), NPAD - 1, jnp.int32)]).reshape(NCHUNKS, CHUNK)
    eps = jnp.broadcast_to(jnp.reshape(eps_k.astype(jnp.float32), (1,)), (16,))

    mesh = plsc.VectorSubcoreMesh(core_axis_name="c", subcore_axis_name="s")
    run = pl.kernel(
        _gin_body,
        out_type=jax.ShapeDtypeStruct((N_NODES, D_FEAT), jnp.float32),
        mesh=mesh,
        compiler_params=pltpu.CompilerParams(use_tc_tiling_on_sc=False),
        scratch_types=[
            pltpu.VMEM_SHARED((NPAD, DH), jnp.float32),
            pltpu.VMEM_SHARED((NPAD, DH), jnp.float32),
            pltpu.VMEM((BLK, DH), jnp.float32),
            pltpu.VMEM((NBUF, CHUNK, DH), jnp.float32),
            pltpu.VMEM((GSZ, CHUNK), jnp.int32),
            pltpu.VMEM((GSZ, CHUNK), jnp.int32),
            pltpu.VMEM((16,), jnp.float32),
        ] + [pltpu.SemaphoreType.DMA] * 8,
    )
    return run(table, srcp, dst3, eps)


# speedup vs baseline: 9.9757x; 1.1791x over previous
import jax
import jax.numpy as jnp
from jax import lax
from jax.experimental import pallas as pl
from jax.experimental.pallas import tpu as pltpu
from jax.experimental.pallas import tpu_sc as plsc

N_NODES = 10000
N_EDGES = 320000
D_FEAT = 128
DH = 64
CHUNK = 128
NS = 16
NC = 2
NPAD = 10240
BLK = 80
NBLK = NPAD // BLK
NOUT = N_NODES // BLK
GSZ = 8
NBUF = 4
GROUPS = 20
CPT = GSZ * GROUPS
NCHUNKS = CPT * NS
E_PAD = NCHUNKS * CHUNK


def _gin_body(node, srcp, dst3, eps, out, tbl, acc, bufb, rows,
              sidx2, didx2, epsv, gs0, gs1, gs2, gs3, ss0, ss1, ss2, ss3,
              is0, is1):
    c = lax.axis_index("c")
    s = lax.axis_index("s")
    gsems = (gs0, gs1, gs2, gs3)
    ssems = (ss0, ss1, ss2, ss3)
    isems = (is0, is1)
    col0 = pl.multiple_of(c * DH, DH)

    def idx_load(j0, par):
        pltpu.async_copy(srcp.at[pl.ds(j0, GSZ)], sidx2.at[par], isems[par])
        pltpu.async_copy(dst3.at[pl.ds(j0, GSZ)], didx2.at[par], isems[par])

    def idx_wait(par):
        pltpu.make_async_copy(srcp.at[pl.ds(0, GSZ)], sidx2.at[par],
                              isems[par]).wait()
        pltpu.make_async_copy(dst3.at[pl.ds(0, GSZ)], didx2.at[par],
                              isems[par]).wait()

    idx_load(s * CPT, 0)

    pltpu.sync_copy(eps, epsv)
    scale = epsv[...] + 1.0

    def srow(r, carry):
        for c4 in range(DH // 16):
            dsl = pl.ds(c4 * 16, 16)
            bufb[r, dsl] = bufb[r, dsl] * scale
        return carry

    def stage(k, carry):
        blk = s + NS * k

        @pl.when(blk < NOUT)
        def _():
            off = blk * BLK
            pltpu.sync_copy(node.at[pl.ds(off, BLK), pl.ds(col0, DH)], bufb)
            pltpu.sync_copy(bufb, tbl.at[pl.ds(off, BLK)])
            lax.fori_loop(0, BLK, srow, 0)
            pltpu.sync_copy(bufb, acc.at[pl.ds(off, BLK)])

        return carry

    lax.fori_loop(0, NBLK // NS, stage, 0)
    plsc.subcore_barrier()

    def pair(gg, carry):
        for par in range(2):
            g = gg * 2 + par
            idx_wait(par)
            jn = jnp.minimum(s * CPT + (g + 1) * GSZ, NCHUNKS - GSZ)
            idx_load(jn, 1 - par)
            sidx8 = sidx2.at[par]
            didx8 = didx2.at[par]
            gcp = {}
            scp = {}
            for q in range(GSZ):
                slot = q % NBUF
                if q >= NBUF:
                    scp[q - NBUF].wait()
                gcp[q] = pltpu.async_copy(
                    tbl.at[sidx8.at[q]], rows.at[slot], gsems[slot])
                if q >= 1:
                    p = q - 1
                    gcp[p].wait()
                    scp[p] = pltpu.async_copy(
                        rows.at[p % NBUF], acc.at[didx8.at[p]],
                        ssems[p % NBUF], add=True)
            gcp[GSZ - 1].wait()
            scp[GSZ - 1] = pltpu.async_copy(
                rows.at[(GSZ - 1) % NBUF], acc.at[didx8.at[GSZ - 1]],
                ssems[(GSZ - 1) % NBUF], add=True)
            for q in range(GSZ - NBUF, GSZ):
                scp[q].wait()
        return carry

    lax.fori_loop(0, GROUPS // 2, pair, 0)
    idx_wait(0)
    plsc.subcore_barrier()

    def fout(k, carry):
        blk = s + NS * k

        @pl.when(blk < NOUT)
        def _():
            off = blk * BLK
            pltpu.sync_copy(acc.at[pl.ds(off, BLK)],
                            out.at[pl.ds(off, BLK), pl.ds(col0, DH)])

        return carry

    lax.fori_loop(0, NBLK // NS, fout, 0)


def kernel(node, edge_index, eps_k):
    epad = E_PAD - N_EDGES
    srcp = jnp.concatenate(
        [edge_index[1],
         jnp.full((epad,), N_NODES, jnp.int32)]).reshape(NCHUNKS, CHUNK)
    dst3 = jnp.concatenate(
        [edge_index[0],
         jnp.full((epad,), NPAD - 1, jnp.int32)]).reshape(NCHUNKS, CHUNK)
    eps = jnp.broadcast_to(jnp.reshape(eps_k.astype(jnp.float32), (1,)), (16,))

    mesh = plsc.VectorSubcoreMesh(core_axis_name="c", subcore_axis_name="s")
    run = pl.kernel(
        _gin_body,
        out_type=jax.ShapeDtypeStruct((N_NODES, D_FEAT), jnp.float32),
        mesh=mesh,
        compiler_params=pltpu.CompilerParams(use_tc_tiling_on_sc=False),
        scratch_types=[
            pltpu.VMEM_SHARED((NPAD, DH), jnp.float32),
            pltpu.VMEM_SHARED((NPAD, DH), jnp.float32),
            pltpu.VMEM((BLK, DH), jnp.float32),
            pltpu.VMEM((NBUF, CHUNK, DH), jnp.float32),
            pltpu.VMEM((2, GSZ, CHUNK), jnp.int32),
            pltpu.VMEM((2, GSZ, CHUNK), jnp.int32),
            pltpu.VMEM((16,), jnp.float32),
        ] + [pltpu.SemaphoreType.DMA] * 10,
    )
    return run(node, srcp, dst3, eps)


# speedup vs baseline: 10.3706x; 1.0396x over previous
import jax
import jax.numpy as jnp
from jax import lax
from jax.experimental import pallas as pl
from jax.experimental.pallas import tpu as pltpu
from jax.experimental.pallas import tpu_sc as plsc

N_NODES = 10000
N_EDGES = 320000
D_FEAT = 128
DH = 64
CHUNK = 128
NS = 16
NC = 2
NPAD = 10240
BLK = 80
NBLK = NPAD // BLK
NOUT = N_NODES // BLK
GSZ = 8
NBUF = 4
GROUPS = 20
CPT = GSZ * GROUPS
NCHUNKS = CPT * NS
E_PAD = NCHUNKS * CHUNK


def _gin_body(node, srcp, dst3, eps, out, tbl, acc, bufb, rows,
              sidx2, didx2, epsv, gs0, gs1, gs2, gs3, ss0, ss1, ss2, ss3,
              is0, is1):
    c = lax.axis_index("c")
    s = lax.axis_index("s")
    gsems = (gs0, gs1, gs2, gs3)
    ssems = (ss0, ss1, ss2, ss3)
    isems = (is0, is1)
    col0 = pl.multiple_of(c * DH, DH)

    def idx_load(j0, par):
        pltpu.async_copy(srcp.at[pl.ds(j0, GSZ)], sidx2.at[par], isems[par])
        pltpu.async_copy(dst3.at[pl.ds(j0, GSZ)], didx2.at[par], isems[par])

    def idx_wait(par):
        pltpu.make_async_copy(srcp.at[pl.ds(0, GSZ)], sidx2.at[par],
                              isems[par]).wait()
        pltpu.make_async_copy(dst3.at[pl.ds(0, GSZ)], didx2.at[par],
                              isems[par]).wait()

    idx_load(s * CPT, 0)

    pltpu.sync_copy(eps, epsv)
    scale = epsv[...] + 1.0

    def srow(r, carry):
        for c4 in range(DH // 16):
            dsl = pl.ds(c4 * 16, 16)
            bufb[r, dsl] = bufb[r, dsl] * scale
        return carry

    def stage(k, carry):
        blk = s + NS * k

        @pl.when(blk < NOUT)
        def _():
            off = blk * BLK
            pltpu.sync_copy(node.at[pl.ds(off, BLK), pl.ds(col0, DH)], bufb)
            pltpu.sync_copy(bufb, tbl.at[pl.ds(off, BLK)])
            lax.fori_loop(0, BLK, srow, 0)
            pltpu.sync_copy(bufb, acc.at[pl.ds(off, BLK)])

        return carry

    lax.fori_loop(0, NBLK // NS, stage, 0)
    plsc.subcore_barrier()

    def wait_scatter(slot, par):
        pltpu.make_async_copy(rows.at[slot], acc.at[didx2.at[par, 0]],
                              ssems[slot]).wait()

    def pair(gg, carry):
        for par in range(2):
            g = gg * 2 + par
            idx_wait(par)
            jn = jnp.minimum(s * CPT + (g + 1) * GSZ, NCHUNKS - GSZ)
            idx_load(jn, 1 - par)
            sidx8 = sidx2.at[par]
            didx8 = didx2.at[par]
            gcp = {}
            for q in range(GSZ):
                slot = q % NBUF
                if q < NBUF and par == 0:
                    @pl.when(gg >= 1)
                    def _():
                        wait_scatter(slot, par)
                else:
                    wait_scatter(slot, par)
                gcp[q] = pltpu.async_copy(
                    tbl.at[sidx8.at[q]], rows.at[slot], gsems[slot])
                if q >= 1:
                    p = q - 1
                    gcp[p].wait()
                    pltpu.async_copy(
                        rows.at[p % NBUF], acc.at[didx8.at[p]],
                        ssems[p % NBUF], add=True)
            gcp[GSZ - 1].wait()
            pltpu.async_copy(
                rows.at[(GSZ - 1) % NBUF], acc.at[didx8.at[GSZ - 1]],
                ssems[(GSZ - 1) % NBUF], add=True)
        return carry

    lax.fori_loop(0, GROUPS // 2, pair, 0)
    for slot in range(NBUF):
        wait_scatter(slot, 0)
    idx_wait(0)
    plsc.subcore_barrier()

    def fout(k, carry):
        blk = s + NS * k

        @pl.when(blk < NOUT)
        def _():
            off = blk * BLK
            pltpu.sync_copy(acc.at[pl.ds(off, BLK)],
                            out.at[pl.ds(off, BLK), pl.ds(col0, DH)])

        return carry

    lax.fori_loop(0, NBLK // NS, fout, 0)


def kernel(node, edge_index, eps_k):
    epad = E_PAD - N_EDGES
    srcp = jnp.concatenate(
        [edge_index[1],
         jnp.full((epad,), N_NODES, jnp.int32)]).reshape(NCHUNKS, CHUNK)
    dst3 = jnp.concatenate(
        [edge_index[0],
         jnp.full((epad,), NPAD - 1, jnp.int32)]).reshape(NCHUNKS, CHUNK)
    eps = jnp.broadcast_to(jnp.reshape(eps_k.astype(jnp.float32), (1,)), (16,))

    mesh = plsc.VectorSubcoreMesh(core_axis_name="c", subcore_axis_name="s")
    run = pl.kernel(
        _gin_body,
        out_type=jax.ShapeDtypeStruct((N_NODES, D_FEAT), jnp.float32),
        mesh=mesh,
        compiler_params=pltpu.CompilerParams(use_tc_tiling_on_sc=False),
        scratch_types=[
            pltpu.VMEM_SHARED((NPAD, DH), jnp.float32),
            pltpu.VMEM_SHARED((NPAD, DH), jnp.float32),
            pltpu.VMEM((BLK, DH), jnp.float32),
            pltpu.VMEM((NBUF, CHUNK, DH), jnp.float32),
            pltpu.VMEM((2, GSZ, CHUNK), jnp.int32),
            pltpu.VMEM((2, GSZ, CHUNK), jnp.int32),
            pltpu.VMEM((16,), jnp.float32),
        ] + [pltpu.SemaphoreType.DMA] * 10,
    )
    return run(node, srcp, dst3, eps)
